# baseline (speedup 1.0000x reference)
import functools

import jax
import jax.numpy as jnp
from jax import lax
from jax.experimental import pallas as pl
from jax.experimental.pallas import tpu as pltpu
from jax.experimental.pallas import tpu_sc as plsc

B, N, H = 32, 128, 32
NP = N + 1
JP = 144
NG = JP // 16
VS = 512
VE = 1537
RB = 8
NBLK = 17

_mesh = plsc.VectorSubcoreMesh(core_axis_name="c", subcore_axis_name="s")


@functools.partial(
    pl.kernel,
    mesh=_mesh,
    compiler_params=pltpu.CompilerParams(use_tc_tiling_on_sc=False,
                                         needs_layout_passes=False),
    out_type=jax.ShapeDtypeStruct((B, H, NP, NP), jnp.float32),
    scratch_types=[
        pltpu.VMEM(((VS + 1) * H,), jnp.float32),
        pltpu.VMEM(((VE + 1) * H,), jnp.float32),
        pltpu.VMEM((RB, JP), jnp.int32),
        pltpu.VMEM((3, RB, JP), jnp.int32),
        pltpu.VMEM((RB, JP), jnp.float32),
        pltpu.VMEM((H, RB, NP), jnp.float32),
        pltpu.SemaphoreType.DMA,
    ],
)
def _graph_attn_bias_sc(sp_h, ed_h, spi_h, ei_h, bias_h, out_h,
                        sp_tab, ed_tab, spi, ei, brow, obuf, sem):
    b = lax.axis_index("s") * 2 + lax.axis_index("c")
    pltpu.async_copy(sp_h, sp_tab, sem).wait()
    pltpu.async_copy(ed_h, ed_tab, sem).wait()
    c128 = jnp.full((16,), N, jnp.int32)
    hv = jax.lax.iota(jnp.int32, 16)

    def blk_body(blk, carry):
        r0 = jnp.minimum(blk * RB, NP - RB)
        c1 = pltpu.async_copy(spi_h.at[b, pl.ds(r0, RB)], spi, sem)
        c2 = pltpu.async_copy(ei_h.at[b, :, pl.ds(r0, RB)], ei, sem)
        c3 = pltpu.async_copy(bias_h.at[b, pl.ds(r0, RB)], brow, sem)
        c1.wait()
        c2.wait()
        c3.wait()

        def row_body(rr, c2):
            for jg in range(N // 16):
                sl = pl.ds(jg * 16, 16)
                a_sp = jnp.clip(spi[rr, sl], 0, VS) * H
                a_e0 = jnp.clip(ei[0, rr, sl], 0, VE) * H
                a_e1 = jnp.clip(ei[1, rr, sl], 0, VE) * H
                a_e2 = jnp.clip(ei[2, rr, sl], 0, VE) * H
                bv = brow[rr, sl]

                def h_body(h, c, sl=sl, bv=bv, rr=rr):
                    asp, a0, a1, a2 = c
                    v = plsc.load_gather(sp_tab, [asp])
                    e = (plsc.load_gather(ed_tab, [a0])
                         + plsc.load_gather(ed_tab, [a1])
                         + plsc.load_gather(ed_tab, [a2]))
                    obuf[h, rr, sl] = bv + v + e * (1.0 / 3.0)
                    return (asp + 1, a0 + 1, a1 + 1, a2 + 1)

                lax.fori_loop(0, H, h_body, (a_sp, a_e0, a_e1, a_e2),
                              unroll=True)
            rrv = jnp.zeros((16,), jnp.int32) + rr
            isp = jnp.clip(plsc.load_gather(spi, [rrv, c128]), 0, VS) * H
            ie0 = jnp.clip(plsc.load_gather(
                ei, [jnp.zeros((16,), jnp.int32), rrv, c128]), 0, VE) * H
            ie1 = jnp.clip(plsc.load_gather(
                ei, [jnp.ones((16,), jnp.int32), rrv, c128]), 0, VE) * H
            ie2 = jnp.clip(plsc.load_gather(
                ei, [jnp.full((16,), 2, jnp.int32), rrv, c128]), 0, VE) * H
            bvc = plsc.load_gather(brow, [rrv, c128])
            for g in range(H // 16):
                hg = hv + g * 16
                v = plsc.load_gather(sp_tab, [isp + hg])
                e = (plsc.load_gather(ed_tab, [ie0 + hg])
                     + plsc.load_gather(ed_tab, [ie1 + hg])
                     + plsc.load_gather(ed_tab, [ie2 + hg]))
                plsc.store_scatter(obuf, [hg, rrv, c128],
                                   bvc + v + e * (1.0 / 3.0))
            return c2

        lax.fori_loop(0, RB, row_body, 0)
        pltpu.async_copy(obuf, out_h.at[b, :, pl.ds(r0, RB), :], sem).wait()
        return carry

    lax.fori_loop(0, NBLK, blk_body, 0)


def kernel(attn_bias, spatial_pos, attn_edge_type, spatial_pos_table,
           edge_table, virtual_dist):
    f32 = jnp.float32
    sp_aug = jnp.concatenate(
        [spatial_pos_table.astype(f32), virtual_dist.astype(f32).reshape(1, H)],
        axis=0).reshape(-1)
    ed_aug = jnp.concatenate(
        [edge_table.astype(f32), jnp.zeros((1, H), f32)], axis=0).reshape(-1)
    spi = spatial_pos.astype(jnp.int32)
    spi = jnp.pad(spi, ((0, 0), (1, 0), (1, 0)), constant_values=VS)
    spi = jnp.pad(spi, ((0, 0), (0, 0), (0, JP - NP)), constant_values=0)
    ei = attn_edge_type.astype(jnp.int32).transpose(0, 3, 1, 2)
    ei = jnp.pad(ei, ((0, 0), (0, 0), (1, 0), (1, 0)), constant_values=VE)
    ei = jnp.pad(ei, ((0, 0), (0, 0), (0, 0), (0, JP - NP)), constant_values=0)
    biasp = jnp.pad(attn_bias.astype(f32), ((0, 0), (0, 0), (0, JP - NP)))
    return _graph_attn_bias_sc(sp_aug, ed_aug, spi, ei, biasp)


# speedup vs baseline: 1.3692x; 1.3692x over previous
import functools

import jax
import jax.numpy as jnp
from jax import lax
from jax.experimental import pallas as pl
from jax.experimental.pallas import tpu as pltpu
from jax.experimental.pallas import tpu_sc as plsc

B, N, H = 32, 128, 32
NP = N + 1
JP = 144
NG = JP // 16
VS = 512
VE = 1537
RB = 8
NBLK = 17

_mesh = plsc.VectorSubcoreMesh(core_axis_name="c", subcore_axis_name="s")


@functools.partial(
    pl.kernel,
    mesh=_mesh,
    compiler_params=pltpu.CompilerParams(use_tc_tiling_on_sc=False,
                                         needs_layout_passes=False),
    out_type=jax.ShapeDtypeStruct((B, H, NP, NP), jnp.float32),
    scratch_types=[
        pltpu.VMEM(((VS + 1) * H,), jnp.float32),
        pltpu.VMEM(((VE + 1) * H,), jnp.float32),
        pltpu.VMEM((RB, JP), jnp.int32),
        pltpu.VMEM((3, RB, JP), jnp.int32),
        pltpu.VMEM((RB, JP), jnp.float32),
        pltpu.VMEM((H, RB, NP), jnp.float32),
        pltpu.SemaphoreType.DMA,
    ],
)
def _graph_attn_bias_sc(sp_h, ed_h, spi_h, ei_h, bias_h, out_h,
                        sp_tab, ed_tab, spi, ei, brow, obuf, sem):
    b = lax.axis_index("s") * 2 + lax.axis_index("c")
    pltpu.async_copy(sp_h, sp_tab, sem).wait()
    pltpu.async_copy(ed_h, ed_tab, sem).wait()
    c128 = jnp.full((16,), N, jnp.int32)
    hv = jax.lax.iota(jnp.int32, 16)

    def blk_body(blk, carry):
        r0 = jnp.minimum(blk * RB, NP - RB)
        c1 = pltpu.async_copy(spi_h.at[b, pl.ds(r0, RB)], spi, sem)
        c2 = pltpu.async_copy(ei_h.at[b, :, pl.ds(r0, RB)], ei, sem)
        c3 = pltpu.async_copy(bias_h.at[b, pl.ds(r0, RB)], brow, sem)
        c1.wait()
        c2.wait()
        c3.wait()

        def row_body(rr, c2):
            for jg in range(0, N // 16, 2):
                sls = [pl.ds((jg + u) * 16, 16) for u in range(2)]
                addrs = []
                bvs = []
                for sl in sls:
                    addrs += [jnp.clip(spi[rr, sl], 0, VS) * H,
                              jnp.clip(ei[0, rr, sl], 0, VE) * H,
                              jnp.clip(ei[1, rr, sl], 0, VE) * H,
                              jnp.clip(ei[2, rr, sl], 0, VE) * H]
                    bvs.append(brow[rr, sl])

                @plsc.parallel_loop(0, H, 1, unroll=4)
                def _h_loop(h, addrs=addrs, sls=sls, bvs=bvs, rr=rr):
                    for u, sl in enumerate(sls):
                        asp, a0, a1, a2 = (a + h for a in addrs[4 * u:
                                                                4 * u + 4])
                        v = plsc.load_gather(sp_tab, [asp])
                        e = (plsc.load_gather(ed_tab, [a0])
                             + plsc.load_gather(ed_tab, [a1])
                             + plsc.load_gather(ed_tab, [a2]))
                        obuf[h, rr, sl] = bvs[u] + v + e * (1.0 / 3.0)
            rrv = jnp.zeros((16,), jnp.int32) + rr
            isp = jnp.clip(plsc.load_gather(spi, [rrv, c128]), 0, VS) * H
            ie0 = jnp.clip(plsc.load_gather(
                ei, [jnp.zeros((16,), jnp.int32), rrv, c128]), 0, VE) * H
            ie1 = jnp.clip(plsc.load_gather(
                ei, [jnp.ones((16,), jnp.int32), rrv, c128]), 0, VE) * H
            ie2 = jnp.clip(plsc.load_gather(
                ei, [jnp.full((16,), 2, jnp.int32), rrv, c128]), 0, VE) * H
            bvc = plsc.load_gather(brow, [rrv, c128])
            for g in range(H // 16):
                hg = hv + g * 16
                v = plsc.load_gather(sp_tab, [isp + hg])
                e = (plsc.load_gather(ed_tab, [ie0 + hg])
                     + plsc.load_gather(ed_tab, [ie1 + hg])
                     + plsc.load_gather(ed_tab, [ie2 + hg]))
                plsc.store_scatter(obuf, [hg, rrv, c128],
                                   bvc + v + e * (1.0 / 3.0))
            return c2

        lax.fori_loop(0, RB, row_body, 0)
        pltpu.async_copy(obuf, out_h.at[b, :, pl.ds(r0, RB), :], sem).wait()
        return carry

    lax.fori_loop(0, NBLK, blk_body, 0)


def kernel(attn_bias, spatial_pos, attn_edge_type, spatial_pos_table,
           edge_table, virtual_dist):
    f32 = jnp.float32
    sp_aug = jnp.concatenate(
        [spatial_pos_table.astype(f32), virtual_dist.astype(f32).reshape(1, H)],
        axis=0).reshape(-1)
    ed_aug = jnp.concatenate(
        [edge_table.astype(f32), jnp.zeros((1, H), f32)], axis=0).reshape(-1)
    spi = spatial_pos.astype(jnp.int32)
    spi = jnp.pad(spi, ((0, 0), (1, 0), (1, 0)), constant_values=VS)
    spi = jnp.pad(spi, ((0, 0), (0, 0), (0, JP - NP)), constant_values=0)
    ei = attn_edge_type.astype(jnp.int32).transpose(0, 3, 1, 2)
    ei = jnp.pad(ei, ((0, 0), (0, 0), (1, 0), (1, 0)), constant_values=VE)
    ei = jnp.pad(ei, ((0, 0), (0, 0), (0, 0), (0, JP - NP)), constant_values=0)
    biasp = jnp.pad(attn_bias.astype(f32), ((0, 0), (0, 0), (0, JP - NP)))
    return _graph_attn_bias_sc(sp_aug, ed_aug, spi, ei, biasp)


# speedup vs baseline: 3.6727x; 2.6824x over previous
import functools

import jax
import jax.numpy as jnp
from jax import lax
from jax.experimental import pallas as pl
from jax.experimental.pallas import tpu as pltpu
from jax.experimental.pallas import tpu_sc as plsc

B, N, H = 32, 128, 32
NP = N + 1
JP = 144
NG = JP // 16
VS = 512
VE = 1537
RB = 8
NBLK = 17
TS = H + 1

_mesh = plsc.VectorSubcoreMesh(core_axis_name="c", subcore_axis_name="s")


@functools.partial(
    pl.kernel,
    mesh=_mesh,
    compiler_params=pltpu.CompilerParams(use_tc_tiling_on_sc=False,
                                         needs_layout_passes=False),
    out_type=jax.ShapeDtypeStruct((B, H, NP, NP), jnp.float32),
    scratch_types=[
        pltpu.VMEM(((VS + 1) * TS,), jnp.float32),
        pltpu.VMEM(((VE + 1) * TS,), jnp.float32),
        pltpu.VMEM((RB, JP), jnp.int32),
        pltpu.VMEM((3, RB, JP), jnp.int32),
        pltpu.VMEM((RB, JP), jnp.float32),
        pltpu.VMEM((H, RB, NP), jnp.float32),
        pltpu.SemaphoreType.DMA,
    ],
)
def _graph_attn_bias_sc(sp_h, ed_h, spi_h, ei_h, bias_h, out_h,
                        sp_tab, ed_tab, spi, ei, brow, obuf, sem):
    b = lax.axis_index("s") * 2 + lax.axis_index("c")
    pltpu.async_copy(sp_h, sp_tab, sem).wait()
    pltpu.async_copy(ed_h, ed_tab, sem).wait()
    c128 = jnp.full((16,), N, jnp.int32)
    hv = jax.lax.iota(jnp.int32, 16)

    def blk_body(blk, carry):
        r0 = jnp.minimum(blk * RB, NP - RB)
        c1 = pltpu.async_copy(spi_h.at[b, pl.ds(r0, RB)], spi, sem)
        c2 = pltpu.async_copy(ei_h.at[b, :, pl.ds(r0, RB)], ei, sem)
        c3 = pltpu.async_copy(bias_h.at[b, pl.ds(r0, RB)], brow, sem)
        c1.wait()
        c2.wait()
        c3.wait()

        def row_body(rr, c2):
            for jg in range(0, N // 16, 2):
                sls = [pl.ds((jg + u) * 16, 16) for u in range(2)]
                addrs = []
                bvs = []
                for sl in sls:
                    addrs += [jnp.clip(spi[rr, sl], 0, VS) * TS,
                              jnp.clip(ei[0, rr, sl], 0, VE) * TS,
                              jnp.clip(ei[1, rr, sl], 0, VE) * TS,
                              jnp.clip(ei[2, rr, sl], 0, VE) * TS]
                    bvs.append(brow[rr, sl])

                @plsc.parallel_loop(0, H, 1, unroll=4)
                def _h_loop(h, addrs=addrs, sls=sls, bvs=bvs, rr=rr):
                    for u, sl in enumerate(sls):
                        asp, a0, a1, a2 = (a + h for a in addrs[4 * u:
                                                                4 * u + 4])
                        v = plsc.load_gather(sp_tab, [asp])
                        e = (plsc.load_gather(ed_tab, [a0])
                             + plsc.load_gather(ed_tab, [a1])
                             + plsc.load_gather(ed_tab, [a2]))
                        obuf[h, rr, sl] = bvs[u] + v + e * (1.0 / 3.0)
            rrv = jnp.zeros((16,), jnp.int32) + rr
            isp = jnp.clip(plsc.load_gather(spi, [rrv, c128]), 0, VS) * TS
            ie0 = jnp.clip(plsc.load_gather(
                ei, [jnp.zeros((16,), jnp.int32), rrv, c128]), 0, VE) * TS
            ie1 = jnp.clip(plsc.load_gather(
                ei, [jnp.ones((16,), jnp.int32), rrv, c128]), 0, VE) * TS
            ie2 = jnp.clip(plsc.load_gather(
                ei, [jnp.full((16,), 2, jnp.int32), rrv, c128]), 0, VE) * TS
            bvc = plsc.load_gather(brow, [rrv, c128])
            for g in range(H // 16):
                hg = hv + g * 16
                v = plsc.load_gather(sp_tab, [isp + hg])
                e = (plsc.load_gather(ed_tab, [ie0 + hg])
                     + plsc.load_gather(ed_tab, [ie1 + hg])
                     + plsc.load_gather(ed_tab, [ie2 + hg]))
                plsc.store_scatter(obuf, [hg, rrv, c128],
                                   bvc + v + e * (1.0 / 3.0))
            return c2

        lax.fori_loop(0, RB, row_body, 0)
        pltpu.async_copy(obuf, out_h.at[b, :, pl.ds(r0, RB), :], sem).wait()
        return carry

    lax.fori_loop(0, NBLK, blk_body, 0)


def kernel(attn_bias, spatial_pos, attn_edge_type, spatial_pos_table,
           edge_table, virtual_dist):
    f32 = jnp.float32
    sp_aug = jnp.concatenate(
        [spatial_pos_table.astype(f32), virtual_dist.astype(f32).reshape(1, H)],
        axis=0)
    ed_aug = jnp.concatenate(
        [edge_table.astype(f32), jnp.zeros((1, H), f32)], axis=0)
    sp_aug = jnp.pad(sp_aug, ((0, 0), (0, TS - H))).reshape(-1)
    ed_aug = jnp.pad(ed_aug, ((0, 0), (0, TS - H))).reshape(-1)
    spi = spatial_pos.astype(jnp.int32)
    spi = jnp.pad(spi, ((0, 0), (1, 0), (1, 0)), constant_values=VS)
    spi = jnp.pad(spi, ((0, 0), (0, 0), (0, JP - NP)), constant_values=0)
    ei = attn_edge_type.astype(jnp.int32).transpose(0, 3, 1, 2)
    ei = jnp.pad(ei, ((0, 0), (0, 0), (1, 0), (1, 0)), constant_values=VE)
    ei = jnp.pad(ei, ((0, 0), (0, 0), (0, 0), (0, JP - NP)), constant_values=0)
    biasp = jnp.pad(attn_bias.astype(f32), ((0, 0), (0, 0), (0, JP - NP)))
    return _graph_attn_bias_sc(sp_aug, ed_aug, spi, ei, biasp)


# speedup vs baseline: 3.7586x; 1.0234x over previous
import functools

import jax
import jax.numpy as jnp
from jax import lax
from jax.experimental import pallas as pl
from jax.experimental.pallas import tpu as pltpu
from jax.experimental.pallas import tpu_sc as plsc

B, N, H = 32, 128, 32
NP = N + 1
JP = 144
NG = JP // 16
VS = 512
VE = 1537
RB = 8
NBLK = 17
TS = H // 2 + 1

_mesh = plsc.VectorSubcoreMesh(core_axis_name="c", subcore_axis_name="s")


@functools.partial(
    pl.kernel,
    mesh=_mesh,
    compiler_params=pltpu.CompilerParams(use_tc_tiling_on_sc=False,
                                         needs_layout_passes=False),
    out_type=jax.ShapeDtypeStruct((B, H, NP, NP), jnp.float32),
    scratch_types=[
        pltpu.VMEM(((VS + 1) * TS,), jnp.int32),
        pltpu.VMEM(((VE + 1) * TS,), jnp.int32),
        pltpu.VMEM((RB, JP), jnp.int32),
        pltpu.VMEM((3, RB, JP), jnp.int32),
        pltpu.VMEM((RB, JP), jnp.float32),
        pltpu.VMEM((H, RB, NP), jnp.float32),
        pltpu.SemaphoreType.DMA,
    ],
)
def _graph_attn_bias_sc(sp_h, ed_h, spi_h, ei_h, bias_h, out_h,
                        sp_tab, ed_tab, spi, ei, brow, obuf, sem):
    b = lax.axis_index("s") * 2 + lax.axis_index("c")
    pltpu.async_copy(sp_h, sp_tab, sem).wait()
    pltpu.async_copy(ed_h, ed_tab, sem).wait()
    c128 = jnp.full((16,), N, jnp.int32)
    hv = jax.lax.iota(jnp.int32, 16)

    def blk_body(blk, carry):
        r0 = jnp.minimum(blk * RB, NP - RB)
        c1 = pltpu.async_copy(spi_h.at[b, pl.ds(r0, RB)], spi, sem)
        c2 = pltpu.async_copy(ei_h.at[b, :, pl.ds(r0, RB)], ei, sem)
        c3 = pltpu.async_copy(bias_h.at[b, pl.ds(r0, RB)], brow, sem)
        c1.wait()
        c2.wait()
        c3.wait()

        def gload2(tab, a):
            g = plsc.load_gather(tab, [a])
            lo, hi = plsc.unpack(plsc.bitcast(g, jnp.bfloat16),
                                 format=plsc.PackFormat.INTERLEAVED)
            return lo.astype(jnp.float32), hi.astype(jnp.float32)

        def row_body(rr, c2):
            for jg in range(N // 16):
                sl = pl.ds(jg * 16, 16)
                bsp = jnp.clip(spi[rr, sl], 0, VS) * TS
                be0 = jnp.clip(ei[0, rr, sl], 0, VE) * TS
                be1 = jnp.clip(ei[1, rr, sl], 0, VE) * TS
                be2 = jnp.clip(ei[2, rr, sl], 0, VE) * TS
                bv = brow[rr, sl]

                @plsc.parallel_loop(0, H // 2, 1, unroll=4)
                def _h_loop(h, sl=sl, bsp=bsp, be0=be0, be1=be1, be2=be2,
                            bv=bv, rr=rr):
                    vl, vh = gload2(sp_tab, bsp + h)
                    e0l, e0h = gload2(ed_tab, be0 + h)
                    e1l, e1h = gload2(ed_tab, be1 + h)
                    e2l, e2h = gload2(ed_tab, be2 + h)
                    obuf[h, rr, sl] = bv + vl + (e0l + e1l + e2l) * (1.0 / 3.0)
                    obuf[h + H // 2, rr, sl] = (bv + vh
                                                + (e0h + e1h + e2h)
                                                * (1.0 / 3.0))
            rrv = jnp.zeros((16,), jnp.int32) + rr
            isp = jnp.clip(plsc.load_gather(spi, [rrv, c128]), 0, VS) * TS
            ie0 = jnp.clip(plsc.load_gather(
                ei, [jnp.zeros((16,), jnp.int32), rrv, c128]), 0, VE) * TS
            ie1 = jnp.clip(plsc.load_gather(
                ei, [jnp.ones((16,), jnp.int32), rrv, c128]), 0, VE) * TS
            ie2 = jnp.clip(plsc.load_gather(
                ei, [jnp.full((16,), 2, jnp.int32), rrv, c128]), 0, VE) * TS
            bvc = plsc.load_gather(brow, [rrv, c128])
            vl, vh = gload2(sp_tab, isp + hv)
            e0l, e0h = gload2(ed_tab, ie0 + hv)
            e1l, e1h = gload2(ed_tab, ie1 + hv)
            e2l, e2h = gload2(ed_tab, ie2 + hv)
            plsc.store_scatter(obuf, [hv, rrv, c128],
                               bvc + vl + (e0l + e1l + e2l) * (1.0 / 3.0))
            plsc.store_scatter(obuf, [hv + H // 2, rrv, c128],
                               bvc + vh + (e0h + e1h + e2h) * (1.0 / 3.0))
            return c2

        lax.fori_loop(0, RB, row_body, 0)
        pltpu.async_copy(obuf, out_h.at[b, :, pl.ds(r0, RB), :], sem).wait()
        return carry

    lax.fori_loop(0, NBLK, blk_body, 0)


def kernel(attn_bias, spatial_pos, attn_edge_type, spatial_pos_table,
           edge_table, virtual_dist):
    f32 = jnp.float32
    sp_aug = jnp.concatenate(
        [spatial_pos_table.astype(f32), virtual_dist.astype(f32).reshape(1, H)],
        axis=0)
    ed_aug = jnp.concatenate(
        [edge_table.astype(f32), jnp.zeros((1, H), f32)], axis=0)

    def _pack(t):
        bb = jax.lax.bitcast_convert_type(t.astype(jnp.bfloat16), jnp.uint16)
        w = (bb[:, :H // 2].astype(jnp.uint32)
             | (bb[:, H // 2:].astype(jnp.uint32) << 16))
        w = jnp.pad(w, ((0, 0), (0, TS - H // 2)))
        return jax.lax.bitcast_convert_type(w, jnp.int32).reshape(-1)

    sp_aug = _pack(sp_aug)
    ed_aug = _pack(ed_aug)
    spi = spatial_pos.astype(jnp.int32)
    spi = jnp.pad(spi, ((0, 0), (1, 0), (1, 0)), constant_values=VS)
    spi = jnp.pad(spi, ((0, 0), (0, 0), (0, JP - NP)), constant_values=0)
    ei = attn_edge_type.astype(jnp.int32).transpose(0, 3, 1, 2)
    ei = jnp.pad(ei, ((0, 0), (0, 0), (1, 0), (1, 0)), constant_values=VE)
    ei = jnp.pad(ei, ((0, 0), (0, 0), (0, 0), (0, JP - NP)), constant_values=0)
    biasp = jnp.pad(attn_bias.astype(f32), ((0, 0), (0, 0), (0, JP - NP)))
    return _graph_attn_bias_sc(sp_aug, ed_aug, spi, ei, biasp)


# speedup vs baseline: 3.9225x; 1.0436x over previous
import functools

import jax
import jax.numpy as jnp
from jax import lax
from jax.experimental import pallas as pl
from jax.experimental.pallas import tpu as pltpu
from jax.experimental.pallas import tpu_sc as plsc

B, N, H = 32, 128, 32
NP = N + 1
JP = 144
NG = JP // 16
VS = 512
VE = 1537
RB = 8
NBLK = 17
TS = H // 2 + 1

_mesh = plsc.VectorSubcoreMesh(core_axis_name="c", subcore_axis_name="s")


@functools.partial(
    pl.kernel,
    mesh=_mesh,
    compiler_params=pltpu.CompilerParams(use_tc_tiling_on_sc=False,
                                         needs_layout_passes=False),
    out_type=jax.ShapeDtypeStruct((B, H, NP, NP), jnp.float32),
    scratch_types=[
        pltpu.VMEM(((VS + 1) * TS,), jnp.int32),
        pltpu.VMEM(((VE + 1) * TS,), jnp.int32),
        pltpu.VMEM((RB, JP), jnp.int32),
        pltpu.VMEM((3, RB, JP), jnp.int32),
        pltpu.VMEM((RB, JP), jnp.float32),
        pltpu.VMEM((2, H, RB, NP), jnp.float32),
        pltpu.SemaphoreType.DMA,
        pltpu.SemaphoreType.DMA,
    ],
)
def _graph_attn_bias_sc(sp_h, ed_h, spi_h, ei_h, bias_h, out_h,
                        sp_tab, ed_tab, spi, ei, brow, obuf2, sem, osem):
    b = lax.axis_index("s") * 2 + lax.axis_index("c")
    pltpu.async_copy(sp_h, sp_tab, sem).wait()
    pltpu.async_copy(ed_h, ed_tab, sem).wait()
    c128 = jnp.full((16,), N, jnp.int32)
    hv = jax.lax.iota(jnp.int32, 16)

    def blk_body(blk, carry):
        r0 = jnp.minimum(blk * RB, NP - RB)
        p = jnp.bitwise_and(blk, 1)
        obuf = obuf2.at[p]
        c1 = pltpu.async_copy(spi_h.at[b, pl.ds(r0, RB)], spi, sem)
        c2 = pltpu.async_copy(ei_h.at[b, :, pl.ds(r0, RB)], ei, sem)
        c3 = pltpu.async_copy(bias_h.at[b, pl.ds(r0, RB)], brow, sem)
        c1.wait()
        c2.wait()
        c3.wait()

        @pl.when(blk >= 2)
        def _drain_older():
            pltpu.make_async_copy(out_h.at[b, :, pl.ds(0, RB), :],
                                  obuf2.at[p], osem).wait()

        def gload2(tab, a):
            g = plsc.load_gather(tab, [a])
            lo, hi = plsc.unpack(plsc.bitcast(g, jnp.bfloat16),
                                 format=plsc.PackFormat.INTERLEAVED)
            return lo.astype(jnp.float32), hi.astype(jnp.float32)

        def row_body(rr, c2):
            for jg in range(N // 16):
                sl = pl.ds(jg * 16, 16)
                bsp = jnp.clip(spi[rr, sl], 0, VS) * TS
                be0 = jnp.clip(ei[0, rr, sl], 0, VE) * TS
                be1 = jnp.clip(ei[1, rr, sl], 0, VE) * TS
                be2 = jnp.clip(ei[2, rr, sl], 0, VE) * TS
                bv = brow[rr, sl]

                @plsc.parallel_loop(0, H // 2, 1, unroll=4)
                def _h_loop(h, sl=sl, bsp=bsp, be0=be0, be1=be1, be2=be2,
                            bv=bv, rr=rr):
                    vl, vh = gload2(sp_tab, bsp + h)
                    e0l, e0h = gload2(ed_tab, be0 + h)
                    e1l, e1h = gload2(ed_tab, be1 + h)
                    e2l, e2h = gload2(ed_tab, be2 + h)
                    obuf[h, rr, sl] = bv + vl + (e0l + e1l + e2l) * (1.0 / 3.0)
                    obuf[h + H // 2, rr, sl] = (bv + vh
                                                + (e0h + e1h + e2h)
                                                * (1.0 / 3.0))
            rrv = jnp.zeros((16,), jnp.int32) + rr
            isp = jnp.clip(plsc.load_gather(spi, [rrv, c128]), 0, VS) * TS
            ie0 = jnp.clip(plsc.load_gather(
                ei, [jnp.zeros((16,), jnp.int32), rrv, c128]), 0, VE) * TS
            ie1 = jnp.clip(plsc.load_gather(
                ei, [jnp.ones((16,), jnp.int32), rrv, c128]), 0, VE) * TS
            ie2 = jnp.clip(plsc.load_gather(
                ei, [jnp.full((16,), 2, jnp.int32), rrv, c128]), 0, VE) * TS
            bvc = plsc.load_gather(brow, [rrv, c128])
            vl, vh = gload2(sp_tab, isp + hv)
            e0l, e0h = gload2(ed_tab, ie0 + hv)
            e1l, e1h = gload2(ed_tab, ie1 + hv)
            e2l, e2h = gload2(ed_tab, ie2 + hv)
            plsc.store_scatter(obuf, [hv, rrv, c128],
                               bvc + vl + (e0l + e1l + e2l) * (1.0 / 3.0))
            plsc.store_scatter(obuf, [hv + H // 2, rrv, c128],
                               bvc + vh + (e0h + e1h + e2h) * (1.0 / 3.0))
            return c2

        lax.fori_loop(0, RB, row_body, 0)
        pltpu.async_copy(obuf, out_h.at[b, :, pl.ds(r0, RB), :], osem)
        return carry

    lax.fori_loop(0, NBLK, blk_body, 0)
    for _ in range(2):
        pltpu.make_async_copy(out_h.at[b, :, pl.ds(0, RB), :],
                              obuf2.at[0], osem).wait()


def kernel(attn_bias, spatial_pos, attn_edge_type, spatial_pos_table,
           edge_table, virtual_dist):
    f32 = jnp.float32
    sp_aug = jnp.concatenate(
        [spatial_pos_table.astype(f32), virtual_dist.astype(f32).reshape(1, H)],
        axis=0)
    ed_aug = jnp.concatenate(
        [edge_table.astype(f32), jnp.zeros((1, H), f32)], axis=0)

    def _pack(t):
        bb = jax.lax.bitcast_convert_type(t.astype(jnp.bfloat16), jnp.uint16)
        w = (bb[:, :H // 2].astype(jnp.uint32)
             | (bb[:, H // 2:].astype(jnp.uint32) << 16))
        w = jnp.pad(w, ((0, 0), (0, TS - H // 2)))
        return jax.lax.bitcast_convert_type(w, jnp.int32).reshape(-1)

    sp_aug = _pack(sp_aug)
    ed_aug = _pack(ed_aug)
    spi = spatial_pos.astype(jnp.int32)
    spi = jnp.pad(spi, ((0, 0), (1, 0), (1, 0)), constant_values=VS)
    spi = jnp.pad(spi, ((0, 0), (0, 0), (0, JP - NP)), constant_values=0)
    ei = attn_edge_type.astype(jnp.int32).transpose(0, 3, 1, 2)
    ei = jnp.pad(ei, ((0, 0), (0, 0), (1, 0), (1, 0)), constant_values=VE)
    ei = jnp.pad(ei, ((0, 0), (0, 0), (0, 0), (0, JP - NP)), constant_values=0)
    biasp = jnp.pad(attn_bias.astype(f32), ((0, 0), (0, 0), (0, JP - NP)))
    return _graph_attn_bias_sc(sp_aug, ed_aug, spi, ei, biasp)
